# baseline (speedup 1.0000x reference)
import jax
import jax.numpy as jnp
from jax.experimental import pallas as pl
from jax.experimental.pallas import tpu as pltpu

_ISSUE, _DEV, _FILE = 4000, 2000, 4000
_N = _ISSUE + _DEV + _FILE
_D = 128
_LEAKY = 0.1
_NS = 10
_RS = 40
_R = _NS * _RS
_PREC = jax.lax.Precision.HIGHEST
_PREC_BIG = jax.lax.Precision.DEFAULT
_SEGS = ((0, _ISSUE), (_ISSUE, _DEV), (_ISSUE + _DEV, _FILE))


def _lk(x):
    return jnp.where(x >= 0, x, _LEAKY * x)


def _layer_body(first, *refs):
    adj_ks, refs = refs[:_NS], refs[_NS:]
    if first:
        (ih, dh, fh, lat, tem, hyp, latn, s_out, aa_out,
         inner, aa_scr) = refs
        aa_in = aa_scr
    else:
        lat, aa_in, s_in, tem, hyp, s_out, inner = refs
    i = pl.program_id(0)

    @pl.when(i == 0)
    def _prologue():
        if first:
            hs = (ih, dh, fh)
            for s, (st, sz) in enumerate(_SEGS):
                aa_scr[st:st + sz, :] = jnp.dot(
                    lat[st:st + sz, :], hs[s][...], precision=_PREC)
        for s, (st, sz) in enumerate(_SEGS):
            inner[s * _D:(s + 1) * _D, :] = _lk(jax.lax.dot_general(
                aa_in[st:st + sz, :], lat[st:st + sz, :],
                (((0,), (0,)), ((), ())), precision=_PREC))

    t = jnp.concatenate(
        [jnp.dot(a[...], lat[...], precision=_PREC_BIG) for a in adj_ks],
        axis=0)
    t = _lk(t)

    rows = pl.ds(i * _R, _R)
    aa_rows = aa_in[rows, :]
    b0, b1 = _ISSUE // _R, (_ISSUE + _DEV) // _R
    for s, lo, hi in ((0, 0, b0), (1, b0, b1), (2, b1, _N // _R)):
        @pl.when((i >= lo) & (i < hi))
        def _seg(s=s):
            hyp[...] = _lk(jnp.dot(
                aa_rows, inner[s * _D:(s + 1) * _D, :], precision=_PREC))

    ln = t + hyp[...]
    tem[...] = t
    if first:
        latn[...] = ln
        s_out[...] = lat[rows, :] + ln
        aa_out[...] = aa_scr[rows, :]
    else:
        s_out[...] = s_in[...] + ln


def _layer1(adj, lat, ih, dh, fh):
    nb = _N // _R
    row = pl.BlockSpec((_R, _D), lambda i: (i, 0))
    full = pl.BlockSpec((_N, _D), lambda i: (0, 0))
    small = pl.BlockSpec((_D, _D), lambda i: (0, 0))
    adj_specs = [
        pl.BlockSpec((_RS, _N), lambda i, k=k: (_NS * i + k, 0))
        for k in range(_NS)
    ]
    body = lambda *r: _layer_body(True, *r)
    return pl.pallas_call(
        body,
        grid=(nb,),
        in_specs=adj_specs + [small, small, small, full],
        out_specs=[row] * 5,
        out_shape=[jax.ShapeDtypeStruct((_N, _D), jnp.float32)] * 5,
        scratch_shapes=[pltpu.VMEM((3 * _D, _D), jnp.float32),
                        pltpu.VMEM((_N, _D), jnp.float32)],
        compiler_params=pltpu.CompilerParams(
            dimension_semantics=("arbitrary",),
        ),
    )(*([adj] * _NS), ih, dh, fh, lat)


def _layer2(adj, lat, aa, s_in):
    nb = _N // _R
    row = pl.BlockSpec((_R, _D), lambda i: (i, 0))
    full = pl.BlockSpec((_N, _D), lambda i: (0, 0))
    adj_specs = [
        pl.BlockSpec((_RS, _N), lambda i, k=k: (_NS * i + k, 0))
        for k in range(_NS)
    ]
    body = lambda *r: _layer_body(False, *r)
    return pl.pallas_call(
        body,
        grid=(nb,),
        in_specs=adj_specs + [full, full, row],
        out_specs=[row] * 3,
        out_shape=[jax.ShapeDtypeStruct((_N, _D), jnp.float32)] * 3,
        scratch_shapes=[pltpu.VMEM((3 * _D, _D), jnp.float32)],
        compiler_params=pltpu.CompilerParams(
            dimension_semantics=("arbitrary",),
        ),
    )(*([adj] * _NS), lat, aa, s_in)


def kernel(adj, keepRate, iEmbeds, dEmbeds, fEmbeds, iHyper, dHyper, fHyper):
    embeds = jnp.concatenate([iEmbeds, dEmbeds, fEmbeds], axis=0)

    tem1, hyp1, lat1, sum1, aa = _layer1(adj, embeds, iHyper, dHyper, fHyper)
    tem2, hyp2, out = _layer2(adj, lat1, aa, sum1)

    return (out, tem1, tem2, hyp1, hyp2)


# speedup vs baseline: 1.0432x; 1.0432x over previous
import jax
import jax.numpy as jnp
from jax.experimental import pallas as pl
from jax.experimental.pallas import tpu as pltpu

_ISSUE, _DEV, _FILE = 4000, 2000, 4000
_N = _ISSUE + _DEV + _FILE
_D = 128
_LEAKY = 0.1
_NS = 2
_RS = 200
_R = _NS * _RS
_PREC = jax.lax.Precision.HIGHEST
_PREC_BIG = jax.lax.Precision.DEFAULT
_SEGS = ((0, _ISSUE), (_ISSUE, _DEV), (_ISSUE + _DEV, _FILE))


def _lk(x):
    return jnp.where(x >= 0, x, _LEAKY * x)


def _layer_body(first, *refs):
    adj_ks, refs = refs[:_NS], refs[_NS:]
    if first:
        (ih, dh, fh, lat, tem, hyp, latn, s_out, aa_out,
         inner, aa_scr) = refs
        aa_in = aa_scr
    else:
        lat, aa_in, s_in, tem, hyp, s_out, inner = refs
    i = pl.program_id(0)

    @pl.when(i == 0)
    def _prologue():
        if first:
            hs = (ih, dh, fh)
            for s, (st, sz) in enumerate(_SEGS):
                aa_scr[st:st + sz, :] = jnp.dot(
                    lat[st:st + sz, :], hs[s][...], precision=_PREC)
        for s, (st, sz) in enumerate(_SEGS):
            inner[s * _D:(s + 1) * _D, :] = _lk(jax.lax.dot_general(
                aa_in[st:st + sz, :], lat[st:st + sz, :],
                (((0,), (0,)), ((), ())), precision=_PREC))

    t = jnp.concatenate(
        [jnp.dot(a[...], lat[...], precision=_PREC_BIG) for a in adj_ks],
        axis=0)
    t = _lk(t)

    rows = pl.ds(i * _R, _R)
    aa_rows = aa_in[rows, :]
    b0, b1 = _ISSUE // _R, (_ISSUE + _DEV) // _R
    for s, lo, hi in ((0, 0, b0), (1, b0, b1), (2, b1, _N // _R)):
        @pl.when((i >= lo) & (i < hi))
        def _seg(s=s):
            hyp[...] = _lk(jnp.dot(
                aa_rows, inner[s * _D:(s + 1) * _D, :], precision=_PREC))

    ln = t + hyp[...]
    tem[...] = t
    if first:
        latn[...] = ln
        s_out[...] = lat[rows, :] + ln
        aa_out[...] = aa_scr[rows, :]
    else:
        s_out[...] = s_in[...] + ln


def _layer1(adj, lat, ih, dh, fh):
    nb = _N // _R
    row = pl.BlockSpec((_R, _D), lambda i: (i, 0))
    full = pl.BlockSpec((_N, _D), lambda i: (0, 0))
    small = pl.BlockSpec((_D, _D), lambda i: (0, 0))
    adj_specs = [
        pl.BlockSpec((_RS, _N), lambda i, k=k: (_NS * i + k, 0))
        for k in range(_NS)
    ]
    body = lambda *r: _layer_body(True, *r)
    return pl.pallas_call(
        body,
        grid=(nb,),
        in_specs=adj_specs + [small, small, small, full],
        out_specs=[row] * 5,
        out_shape=[jax.ShapeDtypeStruct((_N, _D), jnp.float32)] * 5,
        scratch_shapes=[pltpu.VMEM((3 * _D, _D), jnp.float32),
                        pltpu.VMEM((_N, _D), jnp.float32)],
        compiler_params=pltpu.CompilerParams(
            dimension_semantics=("arbitrary",),
        ),
    )(*([adj] * _NS), ih, dh, fh, lat)


def _layer2(adj, lat, aa, s_in):
    nb = _N // _R
    row = pl.BlockSpec((_R, _D), lambda i: (i, 0))
    full = pl.BlockSpec((_N, _D), lambda i: (0, 0))
    adj_specs = [
        pl.BlockSpec((_RS, _N), lambda i, k=k: (_NS * i + k, 0))
        for k in range(_NS)
    ]
    body = lambda *r: _layer_body(False, *r)
    return pl.pallas_call(
        body,
        grid=(nb,),
        in_specs=adj_specs + [full, full, row],
        out_specs=[row] * 3,
        out_shape=[jax.ShapeDtypeStruct((_N, _D), jnp.float32)] * 3,
        scratch_shapes=[pltpu.VMEM((3 * _D, _D), jnp.float32)],
        compiler_params=pltpu.CompilerParams(
            dimension_semantics=("arbitrary",),
        ),
    )(*([adj] * _NS), lat, aa, s_in)


def kernel(adj, keepRate, iEmbeds, dEmbeds, fEmbeds, iHyper, dHyper, fHyper):
    embeds = jnp.concatenate([iEmbeds, dEmbeds, fEmbeds], axis=0)

    tem1, hyp1, lat1, sum1, aa = _layer1(adj, embeds, iHyper, dHyper, fHyper)
    tem2, hyp2, out = _layer2(adj, lat1, aa, sum1)

    return (out, tem1, tem2, hyp1, hyp2)


# speedup vs baseline: 1.0533x; 1.0097x over previous
import jax
import jax.numpy as jnp
from jax.experimental import pallas as pl
from jax.experimental.pallas import tpu as pltpu

_ISSUE, _DEV, _FILE = 4000, 2000, 4000
_N = _ISSUE + _DEV + _FILE
_D = 128
_LEAKY = 0.1
_NS = 1
_RS = 400
_R = _NS * _RS
_PREC = jax.lax.Precision.HIGHEST
_PREC_BIG = jax.lax.Precision.DEFAULT
_SEGS = ((0, _ISSUE), (_ISSUE, _DEV), (_ISSUE + _DEV, _FILE))


def _lk(x):
    return jnp.where(x >= 0, x, _LEAKY * x)


def _layer_body(first, *refs):
    adj_ks, refs = refs[:_NS], refs[_NS:]
    if first:
        (ih, dh, fh, lat, tem, hyp, latn, s_out, aa_out,
         inner, aa_scr) = refs
        aa_in = aa_scr
    else:
        lat, aa_in, s_in, tem, hyp, s_out, inner = refs
    i = pl.program_id(0)

    @pl.when(i == 0)
    def _prologue():
        if first:
            hs = (ih, dh, fh)
            for s, (st, sz) in enumerate(_SEGS):
                aa_scr[st:st + sz, :] = jnp.dot(
                    lat[st:st + sz, :], hs[s][...], precision=_PREC)
        for s, (st, sz) in enumerate(_SEGS):
            inner[s * _D:(s + 1) * _D, :] = _lk(jax.lax.dot_general(
                aa_in[st:st + sz, :], lat[st:st + sz, :],
                (((0,), (0,)), ((), ())), precision=_PREC))

    t = jnp.concatenate(
        [jnp.dot(a[...], lat[...], precision=_PREC_BIG) for a in adj_ks],
        axis=0)
    t = _lk(t)

    rows = pl.ds(i * _R, _R)
    aa_rows = aa_in[rows, :]
    b0, b1 = _ISSUE // _R, (_ISSUE + _DEV) // _R
    for s, lo, hi in ((0, 0, b0), (1, b0, b1), (2, b1, _N // _R)):
        @pl.when((i >= lo) & (i < hi))
        def _seg(s=s):
            hyp[...] = _lk(jnp.dot(
                aa_rows, inner[s * _D:(s + 1) * _D, :], precision=_PREC))

    ln = t + hyp[...]
    tem[...] = t
    if first:
        latn[...] = ln
        s_out[...] = lat[rows, :] + ln
        aa_out[...] = aa_scr[rows, :]
    else:
        s_out[...] = s_in[...] + ln


def _layer1(adj, lat, ih, dh, fh):
    nb = _N // _R
    row = pl.BlockSpec((_R, _D), lambda i: (i, 0))
    full = pl.BlockSpec((_N, _D), lambda i: (0, 0))
    small = pl.BlockSpec((_D, _D), lambda i: (0, 0))
    adj_specs = [
        pl.BlockSpec((_RS, _N), lambda i, k=k: (_NS * i + k, 0))
        for k in range(_NS)
    ]
    body = lambda *r: _layer_body(True, *r)
    return pl.pallas_call(
        body,
        grid=(nb,),
        in_specs=adj_specs + [small, small, small, full],
        out_specs=[row] * 5,
        out_shape=[jax.ShapeDtypeStruct((_N, _D), jnp.float32)] * 5,
        scratch_shapes=[pltpu.VMEM((3 * _D, _D), jnp.float32),
                        pltpu.VMEM((_N, _D), jnp.float32)],
        compiler_params=pltpu.CompilerParams(
            dimension_semantics=("arbitrary",),
        ),
    )(*([adj] * _NS), ih, dh, fh, lat)


def _layer2(adj, lat, aa, s_in):
    nb = _N // _R
    row = pl.BlockSpec((_R, _D), lambda i: (i, 0))
    full = pl.BlockSpec((_N, _D), lambda i: (0, 0))
    adj_specs = [
        pl.BlockSpec((_RS, _N), lambda i, k=k: (_NS * i + k, 0))
        for k in range(_NS)
    ]
    body = lambda *r: _layer_body(False, *r)
    return pl.pallas_call(
        body,
        grid=(nb,),
        in_specs=adj_specs + [full, full, row],
        out_specs=[row] * 3,
        out_shape=[jax.ShapeDtypeStruct((_N, _D), jnp.float32)] * 3,
        scratch_shapes=[pltpu.VMEM((3 * _D, _D), jnp.float32)],
        compiler_params=pltpu.CompilerParams(
            dimension_semantics=("arbitrary",),
        ),
    )(*([adj] * _NS), lat, aa, s_in)


def kernel(adj, keepRate, iEmbeds, dEmbeds, fEmbeds, iHyper, dHyper, fHyper):
    embeds = jnp.concatenate([iEmbeds, dEmbeds, fEmbeds], axis=0)

    tem1, hyp1, lat1, sum1, aa = _layer1(adj, embeds, iHyper, dHyper, fHyper)
    tem2, hyp2, out = _layer2(adj, lat1, aa, sum1)

    return (out, tem1, tem2, hyp1, hyp2)


# speedup vs baseline: 1.1113x; 1.0551x over previous
import jax
import jax.numpy as jnp
from jax.experimental import pallas as pl
from jax.experimental.pallas import tpu as pltpu

_ISSUE, _DEV, _FILE = 4000, 2000, 4000
_N = _ISSUE + _DEV + _FILE
_D = 128
_LEAKY = 0.1
_NS = 1
_RS = 400
_R = _NS * _RS
_PREC = jax.lax.Precision.DEFAULT
_SEGS = ((0, _ISSUE), (_ISSUE, _DEV), (_ISSUE + _DEV, _FILE))


def _lk(x):
    return jnp.where(x >= 0, x, _LEAKY * x)


def _layer_body(first, *refs):
    adj_ks, refs = refs[:_NS], refs[_NS:]
    if first:
        ih, dh, fh, lat, tem, hyp, latn, inner, aa = refs
        emb = lat
    else:
        ih, dh, fh, emb, lat, tem, hyp, out, inner, aa = refs
    i = pl.program_id(0)

    @pl.when(i == 0)
    def _prologue():
        hs = (ih, dh, fh)
        for s, (st, sz) in enumerate(_SEGS):
            aa[st:st + sz, :] = jnp.dot(
                emb[st:st + sz, :], hs[s][...], precision=_PREC)
        for s, (st, sz) in enumerate(_SEGS):
            inner[s * _D:(s + 1) * _D, :] = _lk(jax.lax.dot_general(
                aa[st:st + sz, :], lat[st:st + sz, :],
                (((0,), (0,)), ((), ())), precision=_PREC))

    t = jnp.concatenate(
        [jnp.dot(a[...], lat[...], precision=_PREC) for a in adj_ks],
        axis=0)
    t = _lk(t)

    rows = pl.ds(i * _R, _R)
    aa_rows = aa[rows, :]
    b0, b1 = _ISSUE // _R, (_ISSUE + _DEV) // _R
    for s, lo, hi in ((0, 0, b0), (1, b0, b1), (2, b1, _N // _R)):
        @pl.when((i >= lo) & (i < hi))
        def _seg(s=s):
            hyp[...] = _lk(jnp.dot(
                aa_rows, inner[s * _D:(s + 1) * _D, :], precision=_PREC))

    ln = t + hyp[...]
    tem[...] = t
    if first:
        latn[...] = ln
    else:
        out[...] = (emb[rows, :] + lat[rows, :]) + ln


def _adj_specs():
    return [
        pl.BlockSpec((_RS, _N), lambda i, k=k: (_NS * i + k, 0))
        for k in range(_NS)
    ]


_ROW = pl.BlockSpec((_R, _D), lambda i: (i, 0))
_FULL = pl.BlockSpec((_N, _D), lambda i: (0, 0))
_SMALL = pl.BlockSpec((_D, _D), lambda i: (0, 0))


def _scratch():
    return [pltpu.VMEM((3 * _D, _D), jnp.float32),
            pltpu.VMEM((_N, _D), jnp.float32)]


def _layer1(adj, embeds, ih, dh, fh):
    body = lambda *r: _layer_body(True, *r)
    return pl.pallas_call(
        body,
        grid=(_N // _R,),
        in_specs=_adj_specs() + [_SMALL, _SMALL, _SMALL, _FULL],
        out_specs=[_ROW] * 3,
        out_shape=[jax.ShapeDtypeStruct((_N, _D), jnp.float32)] * 3,
        scratch_shapes=_scratch(),
        compiler_params=pltpu.CompilerParams(
            dimension_semantics=("arbitrary",),
        ),
    )(*([adj] * _NS), ih, dh, fh, embeds)


def _layer2(adj, embeds, lat1, ih, dh, fh):
    body = lambda *r: _layer_body(False, *r)
    return pl.pallas_call(
        body,
        grid=(_N // _R,),
        in_specs=_adj_specs() + [_SMALL, _SMALL, _SMALL, _FULL, _FULL],
        out_specs=[_ROW] * 3,
        out_shape=[jax.ShapeDtypeStruct((_N, _D), jnp.float32)] * 3,
        scratch_shapes=_scratch(),
        compiler_params=pltpu.CompilerParams(
            dimension_semantics=("arbitrary",),
        ),
    )(*([adj] * _NS), ih, dh, fh, embeds, lat1)


def kernel(adj, keepRate, iEmbeds, dEmbeds, fEmbeds, iHyper, dHyper, fHyper):
    embeds = jnp.concatenate([iEmbeds, dEmbeds, fEmbeds], axis=0)

    tem1, hyp1, lat1 = _layer1(adj, embeds, iHyper, dHyper, fHyper)
    tem2, hyp2, out = _layer2(adj, embeds, lat1, iHyper, dHyper, fHyper)

    return (out, tem1, tem2, hyp1, hyp2)
